# baseline (speedup 1.0000x reference)
import functools

import jax
import jax.numpy as jnp
from jax.experimental import pallas as pl


def _balls_kernel(pos_ref, den_ref, rad_ref):
    x = pos_ref[0:1, :]
    y = pos_ref[1:2, :]
    z = pos_ref[2:3, :]

    zz = z * z
    q1 = (jnp.square(x - 0.5) + jnp.square(y)) + zz
    q2 = (jnp.square(x + 0.5) + jnp.square(y + 0.2)) + zz
    inside = (q1 < 0.3) | (q2 < 0.8)

    m = jnp.where(inside, jnp.float32(1.0), jnp.float32(0.0))
    den_ref[...] = m
    rad_ref[0:1, :] = m
    rad_ref[1:3, :] = jnp.zeros_like(pos_ref[1:3, :])


@jax.jit
def _run(position):
    n = position.shape[0]
    lanes = 32768
    grid = n // lanes
    pos_t = position.T
    den, rad = pl.pallas_call(
        _balls_kernel,
        grid=(grid,),
        in_specs=[pl.BlockSpec((3, lanes), lambda i: (0, i))],
        out_specs=[
            pl.BlockSpec((1, lanes), lambda i: (0, i)),
            pl.BlockSpec((3, lanes), lambda i: (0, i)),
        ],
        out_shape=[
            jax.ShapeDtypeStruct((1, n), jnp.float32),
            jax.ShapeDtypeStruct((3, n), jnp.float32),
        ],
    )(pos_t)
    return den.reshape(n), rad.T


def kernel(position, direction):
    del direction
    return _run(position)


# speedup vs baseline: 1.7029x; 1.7029x over previous
import functools

import jax
import jax.numpy as jnp
from jax.experimental import pallas as pl


def _balls_kernel(pos_ref, den_ref, rad_ref):
    x = pos_ref[0:1, :]
    y = pos_ref[1:2, :]
    z = pos_ref[2:3, :]

    zz = z * z
    q1 = (jnp.square(x - 0.5) + jnp.square(y)) + zz
    q2 = (jnp.square(x + 0.5) + jnp.square(y + 0.2)) + zz
    inside = (q1 < 0.3) | (q2 < 0.8)

    m = jnp.where(inside, jnp.float32(1.0), jnp.float32(0.0))
    den_ref[...] = m
    rad_ref[0:1, :] = m
    rad_ref[1:3, :] = jnp.zeros_like(pos_ref[1:3, :])


@jax.jit
def _run(position):
    n = position.shape[0]
    lanes = 131072
    grid = n // lanes
    pos_t = position.T
    den, rad = pl.pallas_call(
        _balls_kernel,
        grid=(grid,),
        in_specs=[pl.BlockSpec((3, lanes), lambda i: (0, i))],
        out_specs=[
            pl.BlockSpec((1, lanes), lambda i: (0, i)),
            pl.BlockSpec((3, lanes), lambda i: (0, i)),
        ],
        out_shape=[
            jax.ShapeDtypeStruct((1, n), jnp.float32),
            jax.ShapeDtypeStruct((3, n), jnp.float32),
        ],
    )(pos_t)
    return den.reshape(n), rad.T


def kernel(position, direction):
    del direction
    return _run(position)


# speedup vs baseline: 1.8110x; 1.0635x over previous
import functools

import jax
import jax.numpy as jnp
from jax.experimental import pallas as pl


def _balls_kernel(pos_ref, den_ref, rad_ref):
    x = pos_ref[0:1, :]
    y = pos_ref[1:2, :]
    z = pos_ref[2:3, :]

    zz = z * z
    q1 = (jnp.square(x - 0.5) + jnp.square(y)) + zz
    q2 = (jnp.square(x + 0.5) + jnp.square(y + 0.2)) + zz
    inside = (q1 < 0.3) | (q2 < 0.8)

    m = jnp.where(inside, jnp.float32(1.0), jnp.float32(0.0))
    den_ref[...] = m
    rad_ref[0:1, :] = m
    rad_ref[1:3, :] = jnp.zeros_like(pos_ref[1:3, :])


@jax.jit
def _run(position):
    n = position.shape[0]
    lanes = 262144
    grid = n // lanes
    pos_t = position.T
    den, rad = pl.pallas_call(
        _balls_kernel,
        grid=(grid,),
        in_specs=[pl.BlockSpec((3, lanes), lambda i: (0, i))],
        out_specs=[
            pl.BlockSpec((1, lanes), lambda i: (0, i)),
            pl.BlockSpec((3, lanes), lambda i: (0, i)),
        ],
        out_shape=[
            jax.ShapeDtypeStruct((1, n), jnp.float32),
            jax.ShapeDtypeStruct((3, n), jnp.float32),
        ],
    )(pos_t)
    return den.reshape(n), rad.T


def kernel(position, direction):
    del direction
    return _run(position)


# speedup vs baseline: 2.0217x; 1.1163x over previous
import functools

import jax
import jax.numpy as jnp
from jax.experimental import pallas as pl


def _balls_kernel(pos_ref, den_ref, rad_ref):
    x = pos_ref[0:1, :]
    y = pos_ref[1:2, :]
    z = pos_ref[2:3, :]

    zz = z * z
    q1 = (jnp.square(x - 0.5) + jnp.square(y)) + zz
    q2 = (jnp.square(x + 0.5) + jnp.square(y + 0.2)) + zz
    inside = (q1 < 0.3) | (q2 < 0.8)

    m = jnp.where(inside, jnp.float32(1.0), jnp.float32(0.0))
    den_ref[...] = m
    rad_ref[0:1, :] = m
    rad_ref[1:3, :] = jnp.zeros_like(pos_ref[1:3, :])


@jax.jit
def _run(position):
    n = position.shape[0]
    lanes = 524288
    grid = n // lanes
    pos_t = position.T
    den, rad = pl.pallas_call(
        _balls_kernel,
        grid=(grid,),
        in_specs=[pl.BlockSpec((3, lanes), lambda i: (0, i))],
        out_specs=[
            pl.BlockSpec((1, lanes), lambda i: (0, i)),
            pl.BlockSpec((3, lanes), lambda i: (0, i)),
        ],
        out_shape=[
            jax.ShapeDtypeStruct((1, n), jnp.float32),
            jax.ShapeDtypeStruct((3, n), jnp.float32),
        ],
    )(pos_t)
    return den.reshape(n), rad.T


def kernel(position, direction):
    del direction
    return _run(position)
